# Initial kernel scaffold; baseline (speedup 1.0000x reference)
import jax
import jax.numpy as jnp
from jax.experimental import pallas as pl


def kernel(x, centers1, centers2):
    raise NotImplementedError("write your pallas kernel here")


# speedup vs baseline: 2.1707x; 2.1707x over previous
import functools

import jax
import jax.numpy as jnp
from jax.experimental import pallas as pl

_N_PER = 32
_BLOCK = 2048


def _first_argmin(dist, k):
    m = jnp.min(dist, axis=-1, keepdims=True)
    idx = jax.lax.broadcasted_iota(jnp.int32, dist.shape, 1)
    return jnp.min(jnp.where(dist == m, idx, k), axis=-1)


def _cluster_kernel(x_ref, c1t_ref, c2t_ref, out_ref):
    x1 = x_ref[:, :128]
    x2 = x_ref[:, 128:]
    c1t = c1t_ref[...]
    c2t = c2t_ref[...]

    c1_sq = jnp.sum(c1t * c1t, axis=0, keepdims=True)
    c2_sq = jnp.sum(c2t * c2t, axis=0, keepdims=True)

    mm1 = jnp.dot(x1, c1t, preferred_element_type=jnp.float32)
    dist1 = c1_sq - 2.0 * mm1
    outer = _first_argmin(dist1, 256)

    mm2 = jnp.dot(x2, c2t, preferred_element_type=jnp.float32)
    dist2 = c2_sq - 2.0 * mm2
    inner = _first_argmin(dist2, 32)

    out_ref[0, 0, :] = inner + outer * _N_PER


@functools.partial(jax.jit, static_argnames=())
def kernel(x, centers1, centers2):
    n = x.shape[0]
    grid = n // _BLOCK
    out = pl.pallas_call(
        _cluster_kernel,
        grid=(grid,),
        in_specs=[
            pl.BlockSpec((_BLOCK, 256), lambda i: (i, 0)),
            pl.BlockSpec((128, 256), lambda i: (0, 0)),
            pl.BlockSpec((128, _N_PER), lambda i: (0, 0)),
        ],
        out_specs=pl.BlockSpec((1, 1, _BLOCK), lambda i: (i, 0, 0)),
        out_shape=jax.ShapeDtypeStruct((grid, 1, _BLOCK), jnp.int32),
    )(x, centers1.T, centers2.T)
    return out.reshape(n)


# speedup vs baseline: 4.7407x; 2.1839x over previous
import jax
import jax.numpy as jnp
from jax.experimental import pallas as pl

_N_PER = 32
_BLOCK = 2048


def _first_argmin_t(dist, k):
    m = jnp.min(dist, axis=0, keepdims=True)
    idx = jax.lax.broadcasted_iota(jnp.int32, dist.shape, 0)
    return jnp.min(jnp.where(dist == m, idx, k), axis=0, keepdims=True)


def _cluster_kernel(x_ref, c1_ref, c2_ref, out_ref):
    x1 = x_ref[:, :128]
    x2 = x_ref[:, 128:]
    c1 = c1_ref[...]
    c2 = c2_ref[...]

    c1_sq = jnp.sum(c1 * c1, axis=1, keepdims=True)
    c2_sq = jnp.sum(c2 * c2, axis=1, keepdims=True)

    dims = (((1,), (1,)), ((), ()))
    mm1 = jax.lax.dot_general(c1, x1, dims,
                              preferred_element_type=jnp.float32)
    dist1 = c1_sq - 2.0 * mm1
    outer = _first_argmin_t(dist1, 256)

    mm2 = jax.lax.dot_general(c2, x2, dims,
                              preferred_element_type=jnp.float32)
    dist2 = c2_sq - 2.0 * mm2
    inner = _first_argmin_t(dist2, 32)

    out_ref[0] = inner + outer * _N_PER


@jax.jit
def kernel(x, centers1, centers2):
    n = x.shape[0]
    grid = n // _BLOCK
    out = pl.pallas_call(
        _cluster_kernel,
        grid=(grid,),
        in_specs=[
            pl.BlockSpec((_BLOCK, 256), lambda i: (i, 0)),
            pl.BlockSpec((256, 128), lambda i: (0, 0)),
            pl.BlockSpec((_N_PER, 128), lambda i: (0, 0)),
        ],
        out_specs=pl.BlockSpec((1, 1, _BLOCK), lambda i: (i, 0, 0)),
        out_shape=jax.ShapeDtypeStruct((grid, 1, _BLOCK), jnp.int32),
    )(x, centers1, centers2)
    return out.reshape(n)


# speedup vs baseline: 5.7522x; 1.2134x over previous
import jax
import jax.numpy as jnp
from jax.experimental import pallas as pl
from jax.experimental.pallas import tpu as pltpu

_N_PER = 32
_BLOCK = 4096


def _first_argmin_t(dist, k):
    m = jnp.min(dist, axis=0, keepdims=True)
    idx = jax.lax.broadcasted_iota(jnp.int32, dist.shape, 0)
    return jnp.min(jnp.where(dist == m, idx, k), axis=0, keepdims=True)


def _cluster_kernel(x1_ref, x2_ref, c1_ref, c2_ref, out_ref,
                    c1sq_ref, c2sq_ref):
    @pl.when(pl.program_id(0) == 0)
    def _():
        c1 = c1_ref[...]
        c2 = c2_ref[...]
        c1sq_ref[...] = jnp.sum(c1 * c1, axis=1, keepdims=True)
        c2sq_ref[...] = jnp.sum(c2 * c2, axis=1, keepdims=True)

    dims = (((1,), (1,)), ((), ()))
    mm1 = jax.lax.dot_general(c1_ref[...], x1_ref[...], dims,
                              preferred_element_type=jnp.float32)
    dist1 = c1sq_ref[...] - 2.0 * mm1
    outer = _first_argmin_t(dist1, 256)

    mm2 = jax.lax.dot_general(c2_ref[...], x2_ref[...], dims,
                              preferred_element_type=jnp.float32)
    dist2 = c2sq_ref[...] - 2.0 * mm2
    inner = _first_argmin_t(dist2, 32)

    out_ref[0] = inner + outer * _N_PER


@jax.jit
def kernel(x, centers1, centers2):
    n = x.shape[0]
    grid = n // _BLOCK
    out = pl.pallas_call(
        _cluster_kernel,
        grid=(grid,),
        in_specs=[
            pl.BlockSpec((_BLOCK, 128), lambda i: (i, 0)),
            pl.BlockSpec((_BLOCK, 128), lambda i: (i, 1)),
            pl.BlockSpec((256, 128), lambda i: (0, 0)),
            pl.BlockSpec((_N_PER, 128), lambda i: (0, 0)),
        ],
        out_specs=pl.BlockSpec((1, 1, _BLOCK), lambda i: (i, 0, 0)),
        out_shape=jax.ShapeDtypeStruct((grid, 1, _BLOCK), jnp.int32),
        scratch_shapes=[
            pltpu.VMEM((256, 1), jnp.float32),
            pltpu.VMEM((_N_PER, 1), jnp.float32),
        ],
    )(x, x, centers1, centers2)
    return out.reshape(n)


# speedup vs baseline: 6.0899x; 1.0587x over previous
import jax
import jax.numpy as jnp
from jax.experimental import pallas as pl
from jax.experimental.pallas import tpu as pltpu

_N_PER = 32
_BLOCK = 8192


def _first_argmin_t(dist, k):
    m = jnp.min(dist, axis=0, keepdims=True)
    idx = jax.lax.broadcasted_iota(jnp.int32, dist.shape, 0)
    return jnp.min(jnp.where(dist == m, idx, k), axis=0, keepdims=True)


def _cluster_kernel(x1_ref, x2_ref, c1_ref, c2_ref, out_ref,
                    c1sq_ref, c2sq_ref):
    @pl.when(pl.program_id(0) == 0)
    def _():
        c1 = c1_ref[...]
        c2 = c2_ref[...]
        c1sq_ref[...] = jnp.sum(c1 * c1, axis=1, keepdims=True)
        c2sq_ref[...] = jnp.sum(c2 * c2, axis=1, keepdims=True)

    dims = (((1,), (1,)), ((), ()))
    mm1 = jax.lax.dot_general(c1_ref[...], x1_ref[...], dims,
                              preferred_element_type=jnp.float32)
    dist1 = c1sq_ref[...] - 2.0 * mm1
    outer = _first_argmin_t(dist1, 256)

    mm2 = jax.lax.dot_general(c2_ref[...], x2_ref[...], dims,
                              preferred_element_type=jnp.float32)
    dist2 = c2sq_ref[...] - 2.0 * mm2
    inner = _first_argmin_t(dist2, 32)

    out_ref[0] = inner + outer * _N_PER


@jax.jit
def kernel(x, centers1, centers2):
    n = x.shape[0]
    grid = n // _BLOCK
    out = pl.pallas_call(
        _cluster_kernel,
        grid=(grid,),
        in_specs=[
            pl.BlockSpec((_BLOCK, 128), lambda i: (i, 0)),
            pl.BlockSpec((_BLOCK, 128), lambda i: (i, 1)),
            pl.BlockSpec((256, 128), lambda i: (0, 0)),
            pl.BlockSpec((_N_PER, 128), lambda i: (0, 0)),
        ],
        out_specs=pl.BlockSpec((1, 1, _BLOCK), lambda i: (i, 0, 0)),
        out_shape=jax.ShapeDtypeStruct((grid, 1, _BLOCK), jnp.int32),
        scratch_shapes=[
            pltpu.VMEM((256, 1), jnp.float32),
            pltpu.VMEM((_N_PER, 1), jnp.float32),
        ],
    )(x, x, centers1, centers2)
    return out.reshape(n)
